# baseline (speedup 1.0000x reference)
import functools

import jax
import jax.numpy as jnp
from jax import lax
from jax.experimental import pallas as pl
from jax.experimental.pallas import tpu as pltpu
from jax.experimental.pallas import tpu_sc as plsc

OUT_SIZE = 7
SCALE = 0.25
N_ROIS = 1000
P = 49
B, C, H, W = 2, 256, 192, 192
NC, NS = 2, 16
NW = NC * NS
NPTS = N_ROIS * P
PER_TILE = 1536
NPAD = PER_TILE * NW
CHUNK = 32
NCHUNK = PER_TILE // CHUNK
GRP = 16


def _roi_align_sc(table, rois_pad):
    mesh = plsc.VectorSubcoreMesh(core_axis_name="c", subcore_axis_name="s")

    @functools.partial(
        pl.kernel,
        mesh=mesh,
        compiler_params=pltpu.CompilerParams(use_tc_tiling_on_sc=False, needs_layout_passes=False),
        out_type=jax.ShapeDtypeStruct((NPAD * C,), jnp.float32),
        scratch_types=[
            pltpu.VMEM((8192,), jnp.float32),
            pltpu.VMEM((NCHUNK, 4 * CHUNK), jnp.int32),
            pltpu.VMEM((PER_TILE * 4,), jnp.float32),
            pltpu.VMEM((2, 4 * CHUNK, C), jnp.bfloat16),
            pltpu.VMEM((2 * CHUNK * C,), jnp.float32),
            pltpu.SemaphoreType.DMA,
            pltpu.SemaphoreType.DMA,
            pltpu.SemaphoreType.DMA,
            pltpu.SemaphoreType.DMA,
        ],
    )
    def body(table_hbm, rois_hbm, out_hbm, rois_v, idx_v, w_v, rows_v, out_v,
             gsem0, gsem1, osem0, osem1):
        gsem = (gsem0, gsem1)
        osem = (osem0, osem1)
        cid = lax.axis_index("c")
        sid = lax.axis_index("s")
        wid = sid * NC + cid
        base_pt = wid * PER_TILE
        pltpu.sync_copy(rois_hbm, rois_v)

        lane = lax.iota(jnp.int32, GRP)
        lane2 = lane * 2

        def grp_body(ch, _):
            for sub in range(CHUNK // GRP):
                g = ch * (CHUNK // GRP) + sub
                pid = base_pt + g * GRP + lane
                n = jnp.minimum(lax.div(pid, P), N_ROIS - 1)
                p = lax.rem(pid, P)
                ph = lax.div(p, OUT_SIZE)
                pw = lax.rem(p, OUT_SIZE)
                r8 = n * 8
                bi = plsc.load_gather(rois_v, [r8]).astype(jnp.int32)
                x1 = plsc.load_gather(rois_v, [r8 + 1])
                y1 = plsc.load_gather(rois_v, [r8 + 2])
                x2 = plsc.load_gather(rois_v, [r8 + 3])
                y2 = plsc.load_gather(rois_v, [r8 + 4])
                px = (pw.astype(jnp.float32) + 0.5) / float(OUT_SIZE)
                py = (ph.astype(jnp.float32) + 0.5) / float(OUT_SIZE)
                x = (x1 + px * (x2 - x1)) * SCALE - 0.5
                y = (y1 + py * (y2 - y1)) * SCALE - 0.5
                x0 = x.astype(jnp.int32)
                x0 = jnp.where(x0.astype(jnp.float32) > x, x0 - 1, x0)
                y0 = y.astype(jnp.int32)
                y0 = jnp.where(y0.astype(jnp.float32) > y, y0 - 1, y0)
                xb = jnp.clip(x0, 0, W - 2)
                yb = jnp.clip(y0, 0, H - 2)
                xbf = xb.astype(jnp.float32)
                ybf = yb.astype(jnp.float32)
                wx0 = jnp.maximum(0.0, 1.0 - jnp.abs(x - xbf))
                wx1 = jnp.maximum(0.0, 1.0 - jnp.abs(x - (xbf + 1.0)))
                wy0 = jnp.maximum(0.0, 1.0 - jnp.abs(y - ybf))
                wy1 = jnp.maximum(0.0, 1.0 - jnp.abs(y - (ybf + 1.0)))
                base = (bi * H + yb) * W + xb
                taps = (base, base + 1, base + W, base + W + 1)
                wts = (wx0 * wy0, wx1 * wy0, wx0 * wy1, wx1 * wy1)
                row = jnp.full((GRP,), ch, jnp.int32)
                for t in range(4):
                    col = sub * (GRP * 4) + lane * 4 + t
                    plsc.store_scatter(idx_v, [row, col], taps[t])
                    wpos = g * (GRP * 4) + lane * 4 + t
                    plsc.store_scatter(w_v, [wpos], wts[t])
            return 0

        lax.fori_loop(0, NCHUNK, grp_body, 0)

        def start_gather(ch, b):
            pltpu.make_async_copy(
                table_hbm.at[idx_v.at[ch]], rows_v.at[b], gsem[b]).start()

        def wait_gather(ch, b):
            pltpu.make_async_copy(
                table_hbm.at[idx_v.at[ch]], rows_v.at[b], gsem[b]).wait()

        def out_copy(ch, b):
            return pltpu.make_async_copy(
                out_v.at[pl.ds(b * CHUNK * C, CHUNK * C)],
                out_hbm.at[pl.ds((base_pt + ch * CHUNK) * C, CHUNK * C)],
                osem[b])

        start_gather(0, 0)
        start_gather(1, 1)

        def chunk_body(i, _):
            for b in range(2):
                ch = i * 2 + b
                wait_gather(ch, b)

                @pl.when(ch >= 2)
                def _():
                    out_copy(ch - 2, b).wait()

                @plsc.parallel_loop(0, CHUNK, 1, unroll=4)
                def pt_body(j):
                    wbase = (ch * CHUNK + j) * 4
                    w0 = plsc.load_gather(w_v, [jnp.full((GRP,), wbase, jnp.int32)])
                    w1 = plsc.load_gather(w_v, [jnp.full((GRP,), wbase + 1, jnp.int32)])
                    w2 = plsc.load_gather(w_v, [jnp.full((GRP,), wbase + 2, jnp.int32)])
                    w3 = plsc.load_gather(w_v, [jnp.full((GRP,), wbase + 3, jnp.int32)])
                    r = j * 4
                    obase = (b * CHUNK + j) * C + lane2
                    for cc in range(C // 32):
                        sl = pl.ds(cc * 32, 32)
                        a0, b0 = plsc.unpack(rows_v[b, r, sl],
                                             format=plsc.PackFormat.INTERLEAVED)
                        a1, b1 = plsc.unpack(rows_v[b, r + 1, sl],
                                             format=plsc.PackFormat.INTERLEAVED)
                        a2, b2 = plsc.unpack(rows_v[b, r + 2, sl],
                                             format=plsc.PackFormat.INTERLEAVED)
                        a3, b3 = plsc.unpack(rows_v[b, r + 3, sl],
                                             format=plsc.PackFormat.INTERLEAVED)
                        acc_e = w0 * a0 + w1 * a1 + w2 * a2 + w3 * a3
                        acc_o = w0 * b0 + w1 * b1 + w2 * b2 + w3 * b3
                        pos = obase + cc * 32
                        plsc.store_scatter(out_v, [pos], acc_e)
                        plsc.store_scatter(out_v, [pos + 1], acc_o)

                out_copy(ch, b).start()

                @pl.when(ch + 2 < NCHUNK)
                def _():
                    start_gather(ch + 2, b)
            return 0

        lax.fori_loop(0, NCHUNK // 2, chunk_body, 0)
        out_copy(NCHUNK - 2, 0).wait()
        out_copy(NCHUNK - 1, 1).wait()

    return body(table, rois_pad)


def kernel(features, rois):
    table = jnp.transpose(features, (0, 2, 3, 1)).reshape(B * H * W, C).astype(jnp.bfloat16)
    rois_pad = jnp.pad(rois, ((0, 1024 - N_ROIS), (0, 3))).reshape(-1)
    out_flat = _roi_align_sc(table, rois_pad)
    out = out_flat[:NPTS * C].reshape(N_ROIS, P, C)
    return jnp.transpose(out, (0, 2, 1)).reshape(N_ROIS, C, OUT_SIZE, OUT_SIZE)


# speedup vs baseline: 1.0210x; 1.0210x over previous
import functools

import jax
import jax.numpy as jnp
from jax import lax
from jax.experimental import pallas as pl
from jax.experimental.pallas import tpu as pltpu
from jax.experimental.pallas import tpu_sc as plsc

OUT_SIZE = 7
SCALE = 0.25
N_ROIS = 1000
P = 49
B, C, H, W = 2, 256, 192, 192
NC, NS = 2, 16
NW = NC * NS
NPTS = N_ROIS * P
PER_TILE = 1536
NPAD = PER_TILE * NW
CHUNK = 64
NCHUNK = PER_TILE // CHUNK
NROW = PER_TILE * 4 // 128
GRP = 16


def _roi_align_sc(table, rois_pad):
    mesh = plsc.VectorSubcoreMesh(core_axis_name="c", subcore_axis_name="s")

    @functools.partial(
        pl.kernel,
        mesh=mesh,
        compiler_params=pltpu.CompilerParams(use_tc_tiling_on_sc=False, needs_layout_passes=False),
        out_type=jax.ShapeDtypeStruct((NPAD * C,), jnp.float32),
        scratch_types=[
            pltpu.VMEM((8192,), jnp.float32),
            pltpu.VMEM((NROW, 128), jnp.int32),
            pltpu.VMEM((PER_TILE * 4,), jnp.float32),
            pltpu.VMEM((2, 4 * CHUNK, C), jnp.bfloat16),
            pltpu.VMEM((2 * CHUNK * C,), jnp.float32),
            pltpu.SemaphoreType.DMA,
            pltpu.SemaphoreType.DMA,
            pltpu.SemaphoreType.DMA,
            pltpu.SemaphoreType.DMA,
        ],
    )
    def body(table_hbm, rois_hbm, out_hbm, rois_v, idx_v, w_v, rows_v, out_v,
             gsem0, gsem1, osem0, osem1):
        gsem = (gsem0, gsem1)
        osem = (osem0, osem1)
        cid = lax.axis_index("c")
        sid = lax.axis_index("s")
        wid = sid * NC + cid
        base_pt = wid * PER_TILE
        pltpu.sync_copy(rois_hbm, rois_v)

        lane = lax.iota(jnp.int32, GRP)
        lane2 = lane * 2

        def grp_body(ch, _):
            for sub in range(CHUNK // GRP):
                g = ch * (CHUNK // GRP) + sub
                pid = base_pt + g * GRP + lane
                n = jnp.minimum(lax.div(pid, P), N_ROIS - 1)
                p = lax.rem(pid, P)
                ph = lax.div(p, OUT_SIZE)
                pw = lax.rem(p, OUT_SIZE)
                r8 = n * 8
                bi = plsc.load_gather(rois_v, [r8]).astype(jnp.int32)
                x1 = plsc.load_gather(rois_v, [r8 + 1])
                y1 = plsc.load_gather(rois_v, [r8 + 2])
                x2 = plsc.load_gather(rois_v, [r8 + 3])
                y2 = plsc.load_gather(rois_v, [r8 + 4])
                px = (pw.astype(jnp.float32) + 0.5) / float(OUT_SIZE)
                py = (ph.astype(jnp.float32) + 0.5) / float(OUT_SIZE)
                x = (x1 + px * (x2 - x1)) * SCALE - 0.5
                y = (y1 + py * (y2 - y1)) * SCALE - 0.5
                x0 = x.astype(jnp.int32)
                x0 = jnp.where(x0.astype(jnp.float32) > x, x0 - 1, x0)
                y0 = y.astype(jnp.int32)
                y0 = jnp.where(y0.astype(jnp.float32) > y, y0 - 1, y0)
                xb = jnp.clip(x0, 0, W - 2)
                yb = jnp.clip(y0, 0, H - 2)
                xbf = xb.astype(jnp.float32)
                ybf = yb.astype(jnp.float32)
                wx0 = jnp.maximum(0.0, 1.0 - jnp.abs(x - xbf))
                wx1 = jnp.maximum(0.0, 1.0 - jnp.abs(x - (xbf + 1.0)))
                wy0 = jnp.maximum(0.0, 1.0 - jnp.abs(y - ybf))
                wy1 = jnp.maximum(0.0, 1.0 - jnp.abs(y - (ybf + 1.0)))
                base = (bi * H + yb) * W + xb
                taps = (base, base + 1, base + W, base + W + 1)
                wts = (wx0 * wy0, wx1 * wy0, wx0 * wy1, wx1 * wy1)
                row = jnp.full((GRP,), ch, jnp.int32)
                for t in range(4):
                    col = sub * (GRP * 4) + lane * 4 + t
                    plsc.store_scatter(idx_v, [row, col], taps[t])
                    wpos = g * (GRP * 4) + lane * 4 + t
                    plsc.store_scatter(w_v, [wpos], wts[t])
            return 0

        lax.fori_loop(0, NROW, grp_body, 0)

        def gcopy(ch, b, h):
            return pltpu.make_async_copy(
                table_hbm.at[idx_v.at[2 * ch + h]],
                rows_v.at[b, pl.ds(h * 128, 128)], gsem[b])

        def start_gather(ch, b):
            gcopy(ch, b, 0).start()
            gcopy(ch, b, 1).start()

        def wait_gather(ch, b):
            gcopy(ch, b, 0).wait()
            gcopy(ch, b, 1).wait()

        def out_copy(ch, b):
            return pltpu.make_async_copy(
                out_v.at[pl.ds(b * CHUNK * C, CHUNK * C)],
                out_hbm.at[pl.ds((base_pt + ch * CHUNK) * C, CHUNK * C)],
                osem[b])

        start_gather(0, 0)
        start_gather(1, 1)

        def chunk_body(i, _):
            for b in range(2):
                ch = i * 2 + b
                wait_gather(ch, b)

                @pl.when(ch >= 2)
                def _():
                    out_copy(ch - 2, b).wait()

                @plsc.parallel_loop(0, CHUNK, 1, unroll=2)
                def pt_body(j):
                    wbase = (ch * CHUNK + j) * 4
                    w0 = plsc.load_gather(w_v, [jnp.full((GRP,), wbase, jnp.int32)])
                    w1 = plsc.load_gather(w_v, [jnp.full((GRP,), wbase + 1, jnp.int32)])
                    w2 = plsc.load_gather(w_v, [jnp.full((GRP,), wbase + 2, jnp.int32)])
                    w3 = plsc.load_gather(w_v, [jnp.full((GRP,), wbase + 3, jnp.int32)])
                    r = j * 4
                    obase = (b * CHUNK + j) * C + lane2
                    for cc in range(C // 32):
                        sl = pl.ds(cc * 32, 32)
                        a0, b0 = plsc.unpack(rows_v[b, r, sl],
                                             format=plsc.PackFormat.INTERLEAVED)
                        a1, b1 = plsc.unpack(rows_v[b, r + 1, sl],
                                             format=plsc.PackFormat.INTERLEAVED)
                        a2, b2 = plsc.unpack(rows_v[b, r + 2, sl],
                                             format=plsc.PackFormat.INTERLEAVED)
                        a3, b3 = plsc.unpack(rows_v[b, r + 3, sl],
                                             format=plsc.PackFormat.INTERLEAVED)
                        acc_e = w0 * a0 + w1 * a1 + w2 * a2 + w3 * a3
                        acc_o = w0 * b0 + w1 * b1 + w2 * b2 + w3 * b3
                        pos = obase + cc * 32
                        plsc.store_scatter(out_v, [pos], acc_e)
                        plsc.store_scatter(out_v, [pos + 1], acc_o)

                out_copy(ch, b).start()

                @pl.when(ch + 2 < NCHUNK)
                def _():
                    start_gather(ch + 2, b)
            return 0

        lax.fori_loop(0, NCHUNK // 2, chunk_body, 0)
        out_copy(NCHUNK - 2, 0).wait()
        out_copy(NCHUNK - 1, 1).wait()

    return body(table, rois_pad)


def kernel(features, rois):
    table = jnp.transpose(features, (0, 2, 3, 1)).reshape(B * H * W, C).astype(jnp.bfloat16)
    rois_pad = jnp.pad(rois, ((0, 1024 - N_ROIS), (0, 3))).reshape(-1)
    out_flat = _roi_align_sc(table, rois_pad)
    out = out_flat[:NPTS * C].reshape(N_ROIS, P, C)
    return jnp.transpose(out, (0, 2, 1)).reshape(N_ROIS, C, OUT_SIZE, OUT_SIZE)
